# baseline (speedup 1.0000x reference)
import jax
import jax.numpy as jnp
from jax.experimental import pallas as pl
from jax.experimental.pallas import tpu as pltpu

_PAR = pltpu.CompilerParams(dimension_semantics=("parallel",))
_PAR_BIG = pltpu.CompilerParams(dimension_semantics=("parallel",),
                                vmem_limit_bytes=120 * 1024 * 1024)


def _block_rows(n, target):
    best = None
    for b in range(8, min(n, target) + 1, 8):
        if n % b == 0:
            best = b
    return best if best is not None else n


def _bf(a):
    return a.astype(jnp.bfloat16)


def _dot(a, b):
    return jnp.dot(a, b, preferred_element_type=jnp.float32)


def _proj_body(x_ref, wg_ref, bg_ref, w1_ref, b1_ref, g0_ref, h0_ref):
    x = x_ref[...]
    g0_ref[...] = _dot(x, wg_ref[...]) + bg_ref[...]
    h0_ref[...] = _dot(x, w1_ref[...]) + b1_ref[...]


def _hyper_body(hh0_ref, hh1_ref, h0_ref, w2_ref, b2_ref, he_ref, z_ref):
    q = hh0_ref.shape[0]
    h0b = _bf(h0_ref[...])
    for s, hh in enumerate((hh0_ref, hh1_ref)):
        he = jnp.maximum(_dot(_bf(hh[...]), h0b), 0.0)
        he_ref[s * q:(s + 1) * q, :] = he
        z_ref[s * q:(s + 1) * q, :] = _dot(he, w2_ref[...]) + b2_ref[...]


def _main_body(gg0_ref, gg1_ref, hg0_ref, hg1_ref,
               x_ref, g0_ref, z_ref, wm_ref, bm_ref, fused_ref):
    d = x_ref.shape[1]
    q = gg0_ref.shape[0]
    g0b = _bf(g0_ref[...])
    zb = _bf(z_ref[...])
    ggs = (gg0_ref, gg1_ref)
    hgs = (hg0_ref, hg1_ref)
    for s in range(2):
        gnn = jnp.maximum(_dot(_bf(ggs[s][...]), g0b), 0.0)
        hgn = jnp.maximum(_dot(_bf(hgs[s][...]), zb), 0.0)
        fused = _dot(x_ref[s * q:(s + 1) * q, :], wm_ref[0:d, :])
        fused += _dot(gnn, wm_ref[d:2 * d, :])
        fused += _dot(hgn, wm_ref[2 * d:3 * d, :])
        fused_ref[s * q:(s + 1) * q, :] = fused + bm_ref[...]


def kernel(x, norm_GG, norm_HH, norm_HG, Wg, bg, W1, b1, W2, b2, Wm, bm):
    n, d = x.shape
    h = norm_HH.shape[0]
    f32 = jnp.float32
    bg2 = bg.reshape(1, d)
    b12 = b1.reshape(1, d)
    b22 = b2.reshape(1, d)
    bm2 = bm.reshape(1, d)

    bm_n = _block_rows(n, 400)
    bm_h = _block_rows(h, 400)
    bm_main = _block_rows(n, 400)
    qh = bm_h // 2
    qm = bm_main // 2

    g0, h0 = pl.pallas_call(
        _proj_body,
        grid=(n // bm_n,),
        in_specs=[
            pl.BlockSpec((bm_n, d), lambda i: (i, 0)),
            pl.BlockSpec((d, d), lambda i: (0, 0)),
            pl.BlockSpec((1, d), lambda i: (0, 0)),
            pl.BlockSpec((d, d), lambda i: (0, 0)),
            pl.BlockSpec((1, d), lambda i: (0, 0)),
        ],
        out_specs=[
            pl.BlockSpec((bm_n, d), lambda i: (i, 0)),
            pl.BlockSpec((bm_n, d), lambda i: (i, 0)),
        ],
        out_shape=[
            jax.ShapeDtypeStruct((n, d), f32),
            jax.ShapeDtypeStruct((n, d), f32),
        ],
        compiler_params=_PAR,
    )(x, Wg, bg2, W1, b12)

    hyper_emb, z = pl.pallas_call(
        _hyper_body,
        grid=(h // bm_h,),
        in_specs=[
            pl.BlockSpec((qh, n), lambda i: (2 * i, 0)),
            pl.BlockSpec((qh, n), lambda i: (2 * i + 1, 0)),
            pl.BlockSpec((n, d), lambda i: (0, 0)),
            pl.BlockSpec((d, d), lambda i: (0, 0)),
            pl.BlockSpec((1, d), lambda i: (0, 0)),
        ],
        out_specs=[
            pl.BlockSpec((bm_h, d), lambda i: (i, 0)),
            pl.BlockSpec((bm_h, d), lambda i: (i, 0)),
        ],
        out_shape=[
            jax.ShapeDtypeStruct((h, d), f32),
            jax.ShapeDtypeStruct((h, d), f32),
        ],
        compiler_params=_PAR_BIG,
    )(norm_HH, norm_HH, h0, W2, b22)

    fused_emb = pl.pallas_call(
        _main_body,
        grid=(n // bm_main,),
        in_specs=[
            pl.BlockSpec((qm, n), lambda i: (2 * i, 0)),
            pl.BlockSpec((qm, n), lambda i: (2 * i + 1, 0)),
            pl.BlockSpec((qm, h), lambda i: (2 * i, 0)),
            pl.BlockSpec((qm, h), lambda i: (2 * i + 1, 0)),
            pl.BlockSpec((bm_main, d), lambda i: (i, 0)),
            pl.BlockSpec((n, d), lambda i: (0, 0)),
            pl.BlockSpec((h, d), lambda i: (0, 0)),
            pl.BlockSpec((3 * d, d), lambda i: (0, 0)),
            pl.BlockSpec((1, d), lambda i: (0, 0)),
        ],
        out_specs=pl.BlockSpec((bm_main, d), lambda i: (i, 0)),
        out_shape=jax.ShapeDtypeStruct((n, d), f32),
        compiler_params=_PAR_BIG,
    )(norm_GG, norm_GG, norm_HG, norm_HG, x, g0, z, Wm, bm2)

    return (fused_emb, hyper_emb)


# speedup vs baseline: 1.3593x; 1.3593x over previous
import jax
import jax.numpy as jnp
from jax.experimental import pallas as pl
from jax.experimental.pallas import tpu as pltpu

_PAR = pltpu.CompilerParams(dimension_semantics=("parallel",))
_PAR_BIG = pltpu.CompilerParams(dimension_semantics=("parallel",),
                                vmem_limit_bytes=120 * 1024 * 1024)
_SEQ_BIG = pltpu.CompilerParams(dimension_semantics=("arbitrary",),
                                vmem_limit_bytes=120 * 1024 * 1024)


def _block_rows(n, target):
    best = None
    for b in range(8, min(n, target) + 1, 8):
        if n % b == 0:
            best = b
    return best if best is not None else n


def _bf(a):
    return a.astype(jnp.bfloat16)


def _dot(a, b):
    return jnp.dot(a, b, preferred_element_type=jnp.float32)


def _dot_t(a, b):
    return jax.lax.dot_general(a, b, (((0,), (0,)), ((), ())),
                               preferred_element_type=jnp.float32)


def _proj_body(x_ref, wg_ref, bg_ref, w1_ref, b1_ref, g0_ref, h0_ref):
    x = x_ref[...]
    g0_ref[...] = _dot(x, wg_ref[...]) + bg_ref[...]
    h0_ref[...] = _dot(x, w1_ref[...]) + b1_ref[...]


def _hyper_body(hh_ref, h0_ref, w2_ref, b2_ref, he_ref, z_ref):
    he = jnp.maximum(_dot(_bf(hh_ref[...]), _bf(h0_ref[...])), 0.0)
    he_ref[...] = he
    z_ref[...] = _dot(he, w2_ref[...]) + b2_ref[...]


def _hg_body(hgt_ref, z_ref, out_ref, acc_ref):
    k = pl.program_id(0)
    nk = pl.num_programs(0)
    part = _dot_t(_bf(hgt_ref[...]), _bf(z_ref[...]))

    @pl.when(k == 0)
    def _init():
        acc_ref[...] = part

    @pl.when(k != 0)
    def _accum():
        acc_ref[...] += part

    @pl.when(k == nk - 1)
    def _finish():
        out_ref[...] = jnp.maximum(acc_ref[...], 0.0)


def _main_body(gg_ref, x_ref, hgn_ref, g0_ref, wm_ref, bm_ref, fused_ref):
    d = x_ref.shape[1]
    gnn = jnp.maximum(_dot(_bf(gg_ref[...]), _bf(g0_ref[...])), 0.0)
    fused = _dot(x_ref[...], wm_ref[0:d, :])
    fused += _dot(gnn, wm_ref[d:2 * d, :])
    fused += _dot(hgn_ref[...], wm_ref[2 * d:3 * d, :])
    fused_ref[...] = fused + bm_ref[...]


def kernel(x, norm_GG, norm_HH, norm_HG, Wg, bg, W1, b1, W2, b2, Wm, bm):
    n, d = x.shape
    h = norm_HH.shape[0]
    f32 = jnp.float32
    bg2 = bg.reshape(1, d)
    b12 = b1.reshape(1, d)
    b22 = b2.reshape(1, d)
    bm2 = bm.reshape(1, d)
    hgt = jnp.transpose(norm_HG)

    bm_n = _block_rows(n, 2000)
    bm_h = _block_rows(h, 400)
    bm_hg = _block_rows(h, 200)
    bm_main = _block_rows(n, 400)

    g0, h0 = pl.pallas_call(
        _proj_body,
        grid=(n // bm_n,),
        in_specs=[
            pl.BlockSpec((bm_n, d), lambda i: (i, 0)),
            pl.BlockSpec((d, d), lambda i: (0, 0)),
            pl.BlockSpec((1, d), lambda i: (0, 0)),
            pl.BlockSpec((d, d), lambda i: (0, 0)),
            pl.BlockSpec((1, d), lambda i: (0, 0)),
        ],
        out_specs=[
            pl.BlockSpec((bm_n, d), lambda i: (i, 0)),
            pl.BlockSpec((bm_n, d), lambda i: (i, 0)),
        ],
        out_shape=[
            jax.ShapeDtypeStruct((n, d), f32),
            jax.ShapeDtypeStruct((n, d), f32),
        ],
        compiler_params=_PAR,
    )(x, Wg, bg2, W1, b12)

    hyper_emb, z = pl.pallas_call(
        _hyper_body,
        grid=(h // bm_h,),
        in_specs=[
            pl.BlockSpec((bm_h, n), lambda i: (i, 0)),
            pl.BlockSpec((n, d), lambda i: (0, 0)),
            pl.BlockSpec((d, d), lambda i: (0, 0)),
            pl.BlockSpec((1, d), lambda i: (0, 0)),
        ],
        out_specs=[
            pl.BlockSpec((bm_h, d), lambda i: (i, 0)),
            pl.BlockSpec((bm_h, d), lambda i: (i, 0)),
        ],
        out_shape=[
            jax.ShapeDtypeStruct((h, d), f32),
            jax.ShapeDtypeStruct((h, d), f32),
        ],
        compiler_params=_PAR_BIG,
    )(norm_HH, h0, W2, b22)

    hgn = pl.pallas_call(
        _hg_body,
        grid=(h // bm_hg,),
        in_specs=[
            pl.BlockSpec((bm_hg, n), lambda k: (k, 0)),
            pl.BlockSpec((bm_hg, d), lambda k: (k, 0)),
        ],
        out_specs=pl.BlockSpec((n, d), lambda k: (0, 0)),
        out_shape=jax.ShapeDtypeStruct((n, d), f32),
        scratch_shapes=[pltpu.VMEM((n, d), f32)],
        compiler_params=_SEQ_BIG,
    )(hgt, z)

    fused_emb = pl.pallas_call(
        _main_body,
        grid=(n // bm_main,),
        in_specs=[
            pl.BlockSpec((bm_main, n), lambda i: (i, 0)),
            pl.BlockSpec((bm_main, d), lambda i: (i, 0)),
            pl.BlockSpec((bm_main, d), lambda i: (i, 0)),
            pl.BlockSpec((n, d), lambda i: (0, 0)),
            pl.BlockSpec((3 * d, d), lambda i: (0, 0)),
            pl.BlockSpec((1, d), lambda i: (0, 0)),
        ],
        out_specs=pl.BlockSpec((bm_main, d), lambda i: (i, 0)),
        out_shape=jax.ShapeDtypeStruct((n, d), f32),
        compiler_params=_PAR_BIG,
    )(norm_GG, x, hgn, g0, Wm, bm2)

    return (fused_emb, hyper_emb)


# speedup vs baseline: 1.4234x; 1.0471x over previous
import jax
import jax.numpy as jnp
from jax.experimental import pallas as pl
from jax.experimental.pallas import tpu as pltpu

_SEQ_BIG = pltpu.CompilerParams(dimension_semantics=("arbitrary",),
                                vmem_limit_bytes=120 * 1024 * 1024)


def _block_rows(n, target):
    best = None
    for b in range(8, min(n, target) + 1, 8):
        if n % b == 0:
            best = b
    return best if best is not None else n


def _bf(a):
    return a.astype(jnp.bfloat16)


def _dot(a, b):
    return jnp.dot(a, b, preferred_element_type=jnp.float32)


def _dot_t(a, b):
    return jax.lax.dot_general(a, b, (((0,), (0,)), ((), ())),
                               preferred_element_type=jnp.float32)


def _hyper_body(hh_ref, x_ref, w1_ref, b1_ref, w2_ref, b2_ref,
                he_ref, z_ref, h0_ref):
    @pl.when(pl.program_id(0) == 0)
    def _proj():
        h0_ref[...] = _bf(_dot(x_ref[...], w1_ref[...]) + b1_ref[...])

    he = jnp.maximum(_dot(_bf(hh_ref[...]), h0_ref[...]), 0.0)
    he_ref[...] = he
    z_ref[...] = _dot(he, w2_ref[...]) + b2_ref[...]


def _hg_body(hgt_ref, z_ref, out_ref, acc_ref):
    k = pl.program_id(0)
    nk = pl.num_programs(0)
    part = _dot_t(_bf(hgt_ref[...]), _bf(z_ref[...]))

    @pl.when(k == 0)
    def _init():
        acc_ref[...] = part

    @pl.when(k != 0)
    def _accum():
        acc_ref[...] += part

    @pl.when(k == nk - 1)
    def _finish():
        out_ref[...] = jnp.maximum(acc_ref[...], 0.0)


def _main_body(gg_ref, x_ref, hgn_ref, wg_ref, bg_ref, wm_ref, bm_ref,
               fused_ref, g0_ref):
    i = pl.program_id(0)
    bm_blk, d = fused_ref.shape

    @pl.when(i == 0)
    def _proj():
        g0_ref[...] = _bf(_dot(x_ref[...], wg_ref[...]) + bg_ref[...])

    gnn = jnp.maximum(_dot(_bf(gg_ref[...]), g0_ref[...]), 0.0)
    x_blk = x_ref[pl.ds(i * bm_blk, bm_blk), :]
    fused = _dot(x_blk, wm_ref[0:d, :])
    fused += _dot(gnn, wm_ref[d:2 * d, :])
    fused += _dot(hgn_ref[...], wm_ref[2 * d:3 * d, :])
    fused_ref[...] = fused + bm_ref[...]


def kernel(x, norm_GG, norm_HH, norm_HG, Wg, bg, W1, b1, W2, b2, Wm, bm):
    n, d = x.shape
    h = norm_HH.shape[0]
    f32 = jnp.float32
    bf16 = jnp.bfloat16
    bg2 = bg.reshape(1, d)
    b12 = b1.reshape(1, d)
    b22 = b2.reshape(1, d)
    bm2 = bm.reshape(1, d)
    hgt = jnp.transpose(norm_HG)

    bm_h = _block_rows(h, 400)
    bm_hg = _block_rows(h, 200)
    bm_main = _block_rows(n, 400)

    hyper_emb, z = pl.pallas_call(
        _hyper_body,
        grid=(h // bm_h,),
        in_specs=[
            pl.BlockSpec((bm_h, n), lambda i: (i, 0)),
            pl.BlockSpec((n, d), lambda i: (0, 0)),
            pl.BlockSpec((d, d), lambda i: (0, 0)),
            pl.BlockSpec((1, d), lambda i: (0, 0)),
            pl.BlockSpec((d, d), lambda i: (0, 0)),
            pl.BlockSpec((1, d), lambda i: (0, 0)),
        ],
        out_specs=[
            pl.BlockSpec((bm_h, d), lambda i: (i, 0)),
            pl.BlockSpec((bm_h, d), lambda i: (i, 0)),
        ],
        out_shape=[
            jax.ShapeDtypeStruct((h, d), f32),
            jax.ShapeDtypeStruct((h, d), f32),
        ],
        scratch_shapes=[pltpu.VMEM((n, d), bf16)],
        compiler_params=_SEQ_BIG,
    )(norm_HH, x, W1, b12, W2, b22)

    hgn = pl.pallas_call(
        _hg_body,
        grid=(h // bm_hg,),
        in_specs=[
            pl.BlockSpec((bm_hg, n), lambda k: (k, 0)),
            pl.BlockSpec((bm_hg, d), lambda k: (k, 0)),
        ],
        out_specs=pl.BlockSpec((n, d), lambda k: (0, 0)),
        out_shape=jax.ShapeDtypeStruct((n, d), f32),
        scratch_shapes=[pltpu.VMEM((n, d), f32)],
        compiler_params=_SEQ_BIG,
    )(hgt, z)

    fused_emb = pl.pallas_call(
        _main_body,
        grid=(n // bm_main,),
        in_specs=[
            pl.BlockSpec((bm_main, n), lambda i: (i, 0)),
            pl.BlockSpec((n, d), lambda i: (0, 0)),
            pl.BlockSpec((bm_main, d), lambda i: (i, 0)),
            pl.BlockSpec((d, d), lambda i: (0, 0)),
            pl.BlockSpec((1, d), lambda i: (0, 0)),
            pl.BlockSpec((3 * d, d), lambda i: (0, 0)),
            pl.BlockSpec((1, d), lambda i: (0, 0)),
        ],
        out_specs=pl.BlockSpec((bm_main, d), lambda i: (i, 0)),
        out_shape=jax.ShapeDtypeStruct((n, d), f32),
        scratch_shapes=[pltpu.VMEM((n, d), bf16)],
        compiler_params=_SEQ_BIG,
    )(norm_GG, x, hgn, Wg, bg2, Wm, bm2)

    return (fused_emb, hyper_emb)
